# baseline (speedup 1.0000x reference)
import functools

import jax
import jax.numpy as jnp
from jax.experimental import pallas as pl
from jax.experimental.pallas import tpu as pltpu

K = 1024
D = 256
DEPTH = 3
BETA = 0.25

_ROWS = 512

_HI_MASK = -65536


def _split3(C):
    p0 = jax.lax.bitcast_convert_type(
        jax.lax.bitcast_convert_type(C, jnp.int32) & _HI_MASK, jnp.float32)
    t = C - p0
    p1 = jax.lax.bitcast_convert_type(
        jax.lax.bitcast_convert_type(t, jnp.int32) & _HI_MASK, jnp.float32)
    p2 = t - p1
    return p0.astype(jnp.bfloat16), p1.astype(jnp.bfloat16), p2.astype(jnp.bfloat16)


def _prep_body(cb_ref, c2_ref, pieces_ref):
    for s in range(DEPTH):
        C = cb_ref[s]
        c2_ref[s, :] = jnp.sum(C * C, axis=1)
        p0, p1, p2 = _split3(C)
        pieces_ref[s, 0] = p0
        pieces_ref[s, 1] = p1
        pieces_ref[s, 2] = p2


def _vq_body(z_ref, cb_ref, c2_ref, pieces_ref,
             zq_ref, comp_ref, loss_ref, counts_ref):
    z = z_ref[...]
    r = z
    q_sum = jnp.zeros_like(z)
    comp = jnp.zeros((z.shape[0], 1), dtype=jnp.int32)
    iota = jax.lax.broadcasted_iota(jnp.int32, (z.shape[0], K), 1)
    loss_part = jnp.float32(0.0)

    for s in range(DEPTH):
        C = cb_ref[s]
        r2 = jnp.sum(r * r, axis=1, keepdims=True)
        m = jax.lax.dot_general(2.0 * r, C, (((1,), (1,)), ((), ())),
                                preferred_element_type=jnp.float32)
        d2 = (r2 - m) + c2_ref[s, :][None, :]
        mn = jnp.min(d2, axis=1, keepdims=True)
        idx = jnp.min(jnp.where(d2 == mn, iota, K), axis=1,
                      keepdims=True)
        oh = iota == idx
        oh16 = oh.astype(jnp.bfloat16)
        dims = (((1,), (0,)), ((), ()))
        q = jax.lax.dot_general(oh16, pieces_ref[s, 0], dims,
                                preferred_element_type=jnp.float32)
        q = q + jax.lax.dot_general(oh16, pieces_ref[s, 1], dims,
                                    preferred_element_type=jnp.float32)
        q = q + jax.lax.dot_general(oh16, pieces_ref[s, 2], dims,
                                    preferred_element_type=jnp.float32)
        dr = r - q
        loss_part = loss_part + jnp.sum(dr * dr)
        q_sum = q_sum + q
        r = dr
        comp = comp + idx * jnp.int32(K ** s)
        counts_ref[0, s, :] = jnp.sum(oh.astype(jnp.float32), axis=0)

    zq_ref[...] = z + (q_sum - z)
    comp_ref[...] = comp.reshape(1, z.shape[0], 1)
    loss_ref[...] = loss_part.reshape(1, 1, 1)


def _epi_body(loss_parts_ref, counts_ref, loss_ref, perp_ref, *, n_rows):
    loss = jnp.sum(loss_parts_ref[...]) * (BETA / (n_rows * D))
    loss_ref[...] = loss.reshape(1, 1)
    counts = jnp.sum(counts_ref[...], axis=0)
    probs = counts * (1.0 / n_rows)
    ent = -jnp.sum(probs * jnp.log(probs + 1e-10), axis=1)
    perp = jnp.sum(jnp.exp(ent)) / jnp.float32(DEPTH)
    perp_ref[...] = perp.reshape(1, 1)


@jax.jit
def kernel(z, codebooks):
    B, L, Dd = z.shape
    n = B * L
    flat = z.reshape(n, Dd)
    nblocks = n // _ROWS

    c2, pieces = pl.pallas_call(
        _prep_body,
        out_shape=[
            jax.ShapeDtypeStruct((DEPTH, K), jnp.float32),
            jax.ShapeDtypeStruct((DEPTH, 3, K, Dd), jnp.bfloat16),
        ],
    )(codebooks)

    zq, comp, loss_parts, counts = pl.pallas_call(
        _vq_body,
        grid=(nblocks,),
        in_specs=[
            pl.BlockSpec((_ROWS, Dd), lambda i: (i, 0)),
            pl.BlockSpec((DEPTH, K, Dd), lambda i: (0, 0, 0)),
            pl.BlockSpec((DEPTH, K), lambda i: (0, 0)),
            pl.BlockSpec((DEPTH, 3, K, Dd), lambda i: (0, 0, 0, 0)),
        ],
        out_specs=[
            pl.BlockSpec((_ROWS, Dd), lambda i: (i, 0)),
            pl.BlockSpec((1, _ROWS, 1), lambda i: (i, 0, 0)),
            pl.BlockSpec((1, 1, 1), lambda i: (i, 0, 0)),
            pl.BlockSpec((1, DEPTH, K), lambda i: (i, 0, 0)),
        ],
        out_shape=[
            jax.ShapeDtypeStruct((n, Dd), jnp.float32),
            jax.ShapeDtypeStruct((nblocks, _ROWS, 1), jnp.int32),
            jax.ShapeDtypeStruct((nblocks, 1, 1), jnp.float32),
            jax.ShapeDtypeStruct((nblocks, DEPTH, K), jnp.float32),
        ],
        compiler_params=pltpu.CompilerParams(
            dimension_semantics=("parallel",)),
    )(flat, codebooks, c2, pieces)

    epi = functools.partial(_epi_body, n_rows=n)
    loss, perp = pl.pallas_call(
        epi,
        out_shape=[
            jax.ShapeDtypeStruct((1, 1), jnp.float32),
            jax.ShapeDtypeStruct((1, 1), jnp.float32),
        ],
    )(loss_parts, counts)

    z_q_ste = zq.reshape(B, L, Dd)
    composed = comp.reshape(B, L)
    return (z_q_ste, loss[0, 0], composed, perp[0, 0])


# speedup vs baseline: 1.1246x; 1.1246x over previous
import functools

import jax
import jax.numpy as jnp
from jax.experimental import pallas as pl
from jax.experimental.pallas import tpu as pltpu

K = 1024
D = 256
DEPTH = 3
BETA = 0.25

_ROWS = 1024

_HI_MASK = -65536


def _split3(C):
    p0 = jax.lax.bitcast_convert_type(
        jax.lax.bitcast_convert_type(C, jnp.int32) & _HI_MASK, jnp.float32)
    t = C - p0
    p1 = jax.lax.bitcast_convert_type(
        jax.lax.bitcast_convert_type(t, jnp.int32) & _HI_MASK, jnp.float32)
    p2 = t - p1
    return p0.astype(jnp.bfloat16), p1.astype(jnp.bfloat16), p2.astype(jnp.bfloat16)


def _vq_body(z_ref, cb_ref, zq_ref, comp_ref, loss_ref, perp_ref,
             counts_ref, pieces_ref, c2_ref, *, nblocks, n_rows):
    i = pl.program_id(0)

    @pl.when(i == 0)
    def _init():
        loss_ref[...] = jnp.zeros_like(loss_ref)
        counts_ref[...] = jnp.zeros_like(counts_ref)
        for s in range(DEPTH):
            C = cb_ref[s]
            c2_ref[s, :] = jnp.sum(C * C, axis=1)
            p0, p1, p2 = _split3(C)
            pieces_ref[s, 0] = p0
            pieces_ref[s, 1] = p1
            pieces_ref[s, 2] = p2

    z = z_ref[...]
    r = z
    q_sum = jnp.zeros_like(z)
    comp = jnp.zeros((z.shape[0], 1), dtype=jnp.int32)
    iota = jax.lax.broadcasted_iota(jnp.int32, (z.shape[0], K), 1)
    loss_part = jnp.float32(0.0)

    for s in range(DEPTH):
        C = cb_ref[s]
        r2 = jnp.sum(r * r, axis=1, keepdims=True)
        m = jax.lax.dot_general(2.0 * r, C, (((1,), (1,)), ((), ())),
                                preferred_element_type=jnp.float32)
        d2 = (r2 - m) + c2_ref[s, :][None, :]
        mn = jnp.min(d2, axis=1, keepdims=True)
        idx = jnp.min(jnp.where(d2 == mn, iota, K), axis=1,
                      keepdims=True)
        oh = iota == idx
        oh16 = oh.astype(jnp.bfloat16)
        dims = (((1,), (0,)), ((), ()))
        q = jax.lax.dot_general(oh16, pieces_ref[s, 0], dims,
                                preferred_element_type=jnp.float32)
        q = q + jax.lax.dot_general(oh16, pieces_ref[s, 1], dims,
                                    preferred_element_type=jnp.float32)
        q = q + jax.lax.dot_general(oh16, pieces_ref[s, 2], dims,
                                    preferred_element_type=jnp.float32)
        dr = r - q
        loss_part = loss_part + jnp.sum(dr * dr)
        q_sum = q_sum + q
        r = dr
        comp = comp + idx * jnp.int32(K ** s)
        counts_ref[s, :] = counts_ref[s, :] + jnp.sum(oh.astype(jnp.float32),
                                                      axis=0)

    zq_ref[...] = z + (q_sum - z)
    comp_ref[...] = comp.reshape(1, z.shape[0], 1)
    loss_ref[...] = loss_ref[...] + loss_part * (BETA / (n_rows * D))

    @pl.when(i == nblocks - 1)
    def _fini():
        counts = counts_ref[...]
        probs = counts * (1.0 / n_rows)
        ent = -jnp.sum(probs * jnp.log(probs + 1e-10), axis=1)
        perp = jnp.sum(jnp.exp(ent)) / jnp.float32(DEPTH)
        perp_ref[...] = perp.reshape(1, 1)


@jax.jit
def kernel(z, codebooks):
    B, L, Dd = z.shape
    n = B * L
    flat = z.reshape(n, Dd)
    nblocks = n // _ROWS

    body = functools.partial(_vq_body, nblocks=nblocks, n_rows=n)
    zq, comp, loss, perp = pl.pallas_call(
        body,
        grid=(nblocks,),
        in_specs=[
            pl.BlockSpec((_ROWS, Dd), lambda i: (i, 0)),
            pl.BlockSpec((DEPTH, K, Dd), lambda i: (0, 0, 0)),
        ],
        out_specs=[
            pl.BlockSpec((_ROWS, Dd), lambda i: (i, 0)),
            pl.BlockSpec((1, _ROWS, 1), lambda i: (i, 0, 0)),
            pl.BlockSpec((1, 1), lambda i: (0, 0)),
            pl.BlockSpec((1, 1), lambda i: (0, 0)),
        ],
        out_shape=[
            jax.ShapeDtypeStruct((n, Dd), jnp.float32),
            jax.ShapeDtypeStruct((nblocks, _ROWS, 1), jnp.int32),
            jax.ShapeDtypeStruct((1, 1), jnp.float32),
            jax.ShapeDtypeStruct((1, 1), jnp.float32),
        ],
        scratch_shapes=[
            pltpu.VMEM((DEPTH, K), jnp.float32),
            pltpu.VMEM((DEPTH, 3, K, D), jnp.bfloat16),
            pltpu.VMEM((DEPTH, K), jnp.float32),
        ],
    )(flat, codebooks)

    z_q_ste = zq.reshape(B, L, Dd)
    composed = comp.reshape(B, L)
    return (z_q_ste, loss[0, 0], composed, perp[0, 0])


# speedup vs baseline: 1.1526x; 1.0248x over previous
import functools

import jax
import jax.numpy as jnp
from jax.experimental import pallas as pl
from jax.experimental.pallas import tpu as pltpu

K = 1024
D = 256
DEPTH = 3
BETA = 0.25

_ROWS = 2048

_HI_MASK = -65536


def _split3(C):
    p0 = jax.lax.bitcast_convert_type(
        jax.lax.bitcast_convert_type(C, jnp.int32) & _HI_MASK, jnp.float32)
    t = C - p0
    p1 = jax.lax.bitcast_convert_type(
        jax.lax.bitcast_convert_type(t, jnp.int32) & _HI_MASK, jnp.float32)
    p2 = t - p1
    return p0.astype(jnp.bfloat16), p1.astype(jnp.bfloat16), p2.astype(jnp.bfloat16)


def _vq_body(z_ref, cb_ref, zq_ref, comp_ref, loss_ref, perp_ref,
             counts_ref, pieces_ref, c2_ref, *, nblocks, n_rows):
    i = pl.program_id(0)

    @pl.when(i == 0)
    def _init():
        loss_ref[...] = jnp.zeros_like(loss_ref)
        counts_ref[...] = jnp.zeros_like(counts_ref)
        for s in range(DEPTH):
            C = cb_ref[s]
            c2_ref[s, :] = jnp.sum(C * C, axis=1)
            p0, p1, p2 = _split3(C)
            pieces_ref[s, 0] = p0
            pieces_ref[s, 1] = p1
            pieces_ref[s, 2] = p2

    z = z_ref[...]
    r = z
    q_sum = jnp.zeros_like(z)
    comp = jnp.zeros((z.shape[0], 1), dtype=jnp.int32)
    iota = jax.lax.broadcasted_iota(jnp.int32, (z.shape[0], K), 1)
    loss_part = jnp.float32(0.0)

    for s in range(DEPTH):
        C = cb_ref[s]
        r2 = jnp.sum(r * r, axis=1, keepdims=True)
        m = jax.lax.dot_general(2.0 * r, C, (((1,), (1,)), ((), ())),
                                preferred_element_type=jnp.float32)
        d2 = (r2 - m) + c2_ref[s, :][None, :]
        mn = jnp.min(d2, axis=1, keepdims=True)
        idx = jnp.min(jnp.where(d2 == mn, iota, K), axis=1,
                      keepdims=True)
        oh = iota == idx
        oh16 = oh.astype(jnp.bfloat16)
        dims = (((1,), (0,)), ((), ()))
        q = jax.lax.dot_general(oh16, pieces_ref[s, 0], dims,
                                preferred_element_type=jnp.float32)
        q = q + jax.lax.dot_general(oh16, pieces_ref[s, 1], dims,
                                    preferred_element_type=jnp.float32)
        q = q + jax.lax.dot_general(oh16, pieces_ref[s, 2], dims,
                                    preferred_element_type=jnp.float32)
        dr = r - q
        loss_part = loss_part + jnp.sum(dr * dr)
        q_sum = q_sum + q
        r = dr
        comp = comp + idx * jnp.int32(K ** s)
        counts_ref[s, :] = counts_ref[s, :] + jnp.sum(oh.astype(jnp.float32),
                                                      axis=0)

    zq_ref[...] = z + (q_sum - z)
    comp_ref[...] = comp.reshape(1, z.shape[0], 1)
    loss_ref[...] = loss_ref[...] + loss_part * (BETA / (n_rows * D))

    @pl.when(i == nblocks - 1)
    def _fini():
        counts = counts_ref[...]
        probs = counts * (1.0 / n_rows)
        ent = -jnp.sum(probs * jnp.log(probs + 1e-10), axis=1)
        perp = jnp.sum(jnp.exp(ent)) / jnp.float32(DEPTH)
        perp_ref[...] = perp.reshape(1, 1)


@jax.jit
def kernel(z, codebooks):
    B, L, Dd = z.shape
    n = B * L
    flat = z.reshape(n, Dd)
    nblocks = n // _ROWS

    body = functools.partial(_vq_body, nblocks=nblocks, n_rows=n)
    zq, comp, loss, perp = pl.pallas_call(
        body,
        grid=(nblocks,),
        in_specs=[
            pl.BlockSpec((_ROWS, Dd), lambda i: (i, 0)),
            pl.BlockSpec((DEPTH, K, Dd), lambda i: (0, 0, 0)),
        ],
        out_specs=[
            pl.BlockSpec((_ROWS, Dd), lambda i: (i, 0)),
            pl.BlockSpec((1, _ROWS, 1), lambda i: (i, 0, 0)),
            pl.BlockSpec((1, 1), lambda i: (0, 0)),
            pl.BlockSpec((1, 1), lambda i: (0, 0)),
        ],
        out_shape=[
            jax.ShapeDtypeStruct((n, Dd), jnp.float32),
            jax.ShapeDtypeStruct((nblocks, _ROWS, 1), jnp.int32),
            jax.ShapeDtypeStruct((1, 1), jnp.float32),
            jax.ShapeDtypeStruct((1, 1), jnp.float32),
        ],
        scratch_shapes=[
            pltpu.VMEM((DEPTH, K), jnp.float32),
            pltpu.VMEM((DEPTH, 3, K, D), jnp.bfloat16),
            pltpu.VMEM((DEPTH, K), jnp.float32),
        ],
    )(flat, codebooks)

    z_q_ste = zq.reshape(B, L, Dd)
    composed = comp.reshape(B, L)
    return (z_q_ste, loss[0, 0], composed, perp[0, 0])
